# Initial kernel scaffold; baseline (speedup 1.0000x reference)
import jax
import jax.numpy as jnp
from jax.experimental import pallas as pl


def kernel(user_emb, item_emb, edge_index):
    raise NotImplementedError("write your pallas kernel here")


# speedup vs baseline: 5.4008x; 5.4008x over previous
import functools
import jax
import jax.numpy as jnp
from jax import lax
from jax.experimental import pallas as pl
from jax.experimental.pallas import tpu as pltpu
from jax.experimental.pallas import tpu_sc as plsc

N_USERS = 25000
N_ITEMS = 25000
N = N_USERS + N_ITEMS
NP = 50176
E = 800000
D = 64
DH = 32
LAYERS = 3

NSUB = 16
EPS = E // NSUB
K = 125
INNER = 8
MIDC = EPS // (INNER * K)
RPS = NP // NSUB
RC = 56
NCH = RPS // RC


def _rsqrt16(x):
    i = lax.bitcast_convert_type(x, jnp.int32)
    i = jnp.int32(0x5F3759DF) - lax.shift_right_arithmetic(
        i, jnp.full((16,), 1, jnp.int32))
    g = lax.bitcast_convert_type(i, jnp.float32)
    for _ in range(3):
        g = g * (1.5 - 0.5 * x * g * g)
    return g


def _bcast_row(ref1d, r):
    grp = ref1d[pl.ds((r // 16) * 16, 16)]
    sel = jnp.where(lax.iota(jnp.int32, 16) == r % 16, grp, 0.0)
    return jnp.full((16,), jnp.sum(sel), jnp.float32)


def _body(src2, dst2, hhalf, out, tbl,
          agg_sh, nin_sh, nout_sh,
          idx_s, idx_d, grow, agg_v, acc_v, tbl_v, zero2d,
          s1_v, s2_v, degv, ones_v, sem):
    c = lax.axis_index("c")
    s = lax.axis_index("s")
    z16 = jnp.zeros((16,), jnp.float32)

    def _z2(r, _):
        zero2d[r, pl.ds(0, 16)] = z16
        zero2d[r, pl.ds(16, 16)] = z16
        return 0
    lax.fori_loop(0, RC, _z2, 0)

    def _z1(i, _):
        degv[pl.ds(i * 16, 16)] = z16
        return 0
    lax.fori_loop(0, RPS // 16, _z1, 0)

    def _o1(i, _):
        ones_v[pl.ds(i * 16, 16)] = jnp.ones((16,), jnp.float32)
        return 0
    lax.fori_loop(0, 8, _o1, 0)

    nbase = s * RPS
    pltpu.sync_copy(degv, nout_sh.at[pl.ds(nbase, RPS)])
    pltpu.sync_copy(degv, nin_sh.at[pl.ds(nbase, RPS)])
    plsc.subcore_barrier()

    ones_k = ones_v.at[pl.ds(0, K)]

    def _deg_mid(m, _):
        pltpu.sync_copy(src2.at[s, m], idx_s)
        pltpu.sync_copy(dst2.at[s, m], idx_d)

        def _deg_inner(j, _):
            pltpu.sync_copy(ones_k, nout_sh.at[idx_s.at[j]], add=True)
            pltpu.sync_copy(ones_k, nin_sh.at[idx_d.at[j]], add=True)
            return 0
        lax.fori_loop(0, INNER, _deg_inner, 0)
        return 0
    lax.fori_loop(0, MIDC, _deg_mid, 0)
    plsc.subcore_barrier()

    for nref in (nout_sh, nin_sh):
        pltpu.sync_copy(nref.at[pl.ds(nbase, RPS)], degv)

        def _nrm(i, _):
            x = jnp.maximum(degv[pl.ds(i * 16, 16)], 1.0)
            degv[pl.ds(i * 16, 16)] = _rsqrt16(x)
            return 0
        lax.fori_loop(0, RPS // 16, _nrm, 0)
        pltpu.sync_copy(degv, nref.at[pl.ds(nbase, RPS)])

    def _init_chunk(k, _):
        rows0 = nbase + k * RC
        pltpu.sync_copy(hhalf.at[c, pl.ds(rows0, RC)], agg_v)
        pltpu.sync_copy(nout_sh.at[pl.ds(rows0, RC)], s2_v)
        pltpu.sync_copy(agg_v, out.at[c, pl.ds(rows0, RC)])

        def _scale(r, _):
            g = _bcast_row(s2_v, r)
            tbl_v[r, pl.ds(0, 16)] = agg_v[r, pl.ds(0, 16)] * g
            tbl_v[r, pl.ds(16, 16)] = agg_v[r, pl.ds(16, 16)] * g
            return 0
        lax.fori_loop(0, RC, _scale, 0)
        pltpu.sync_copy(tbl_v, tbl.at[c, 0, pl.ds(rows0, RC)])
        return 0
    lax.fori_loop(0, NCH, _init_chunk, 0)

    for layer in range(1, LAYERS + 1):
        tp = (layer - 1) % 2
        tn = layer % 2
        last = layer == LAYERS

        def _zero_chunk(k, _):
            pltpu.sync_copy(zero2d, agg_sh.at[pl.ds(nbase + k * RC, RC)])
            return 0
        lax.fori_loop(0, NCH, _zero_chunk, 0)
        plsc.subcore_barrier()

        tcur = tbl.at[c, tp]

        def _edge_mid(m, _):
            pltpu.sync_copy(src2.at[s, m], idx_s)
            pltpu.sync_copy(dst2.at[s, m], idx_d)

            def _edge_inner(j, _):
                pltpu.async_copy(tcur.at[idx_s.at[j]], grow, sem).wait()
                pltpu.sync_copy(grow, agg_sh.at[idx_d.at[j]], add=True)
                return 0
            lax.fori_loop(0, INNER, _edge_inner, 0)
            return 0
        lax.fori_loop(0, MIDC, _edge_mid, 0)
        plsc.subcore_barrier()

        def _fin_chunk(k, _):
            rows0 = nbase + k * RC
            pltpu.sync_copy(agg_sh.at[pl.ds(rows0, RC)], agg_v)
            pltpu.sync_copy(nin_sh.at[pl.ds(rows0, RC)], s1_v)
            if not last:
                pltpu.sync_copy(nout_sh.at[pl.ds(rows0, RC)], s2_v)
            pltpu.sync_copy(out.at[c, pl.ds(rows0, RC)], acc_v)

            def _fin(r, _):
                g1 = _bcast_row(s1_v, r)
                h0 = agg_v[r, pl.ds(0, 16)] * g1
                h1 = agg_v[r, pl.ds(16, 16)] * g1
                a0 = acc_v[r, pl.ds(0, 16)] + h0
                a1 = acc_v[r, pl.ds(16, 16)] + h1
                if last:
                    a0 = a0 * 0.25
                    a1 = a1 * 0.25
                acc_v[r, pl.ds(0, 16)] = a0
                acc_v[r, pl.ds(16, 16)] = a1
                if not last:
                    g2 = _bcast_row(s2_v, r)
                    tbl_v[r, pl.ds(0, 16)] = h0 * g2
                    tbl_v[r, pl.ds(16, 16)] = h1 * g2
                return 0
            lax.fori_loop(0, RC, _fin, 0)
            pltpu.sync_copy(acc_v, out.at[c, pl.ds(rows0, RC)])
            if not last:
                pltpu.sync_copy(tbl_v, tbl.at[c, tn, pl.ds(rows0, RC)])
            return 0
        lax.fori_loop(0, NCH, _fin_chunk, 0)


@jax.jit
def _lightgcn_sc(src2, dst2, hhalf):
    mesh = plsc.VectorSubcoreMesh(core_axis_name="c", subcore_axis_name="s")
    f32 = jnp.float32
    run = pl.kernel(
        _body,
        mesh=mesh,
        compiler_params=pltpu.CompilerParams(
            needs_layout_passes=False, use_tc_tiling_on_sc=False),
        out_type=[
            jax.ShapeDtypeStruct((2, NP, DH), f32),
            jax.ShapeDtypeStruct((2, 2, NP, DH), f32),
        ],
        scratch_types=[
            pltpu.VMEM_SHARED((NP, DH), f32),
            pltpu.VMEM_SHARED((NP,), f32),
            pltpu.VMEM_SHARED((NP,), f32),
            pltpu.VMEM((INNER, K), jnp.int32),
            pltpu.VMEM((INNER, K), jnp.int32),
            pltpu.VMEM((K, DH), f32),
            pltpu.VMEM((RC, DH), f32),
            pltpu.VMEM((RC, DH), f32),
            pltpu.VMEM((RC, DH), f32),
            pltpu.VMEM((RC, DH), f32),
            pltpu.VMEM((RC,), f32),
            pltpu.VMEM((RC,), f32),
            pltpu.VMEM((RPS,), f32),
            pltpu.VMEM((128,), f32),
            pltpu.SemaphoreType.DMA,
        ],
    )
    out, _ = run(src2, dst2, hhalf)
    return out


def kernel(user_emb, item_emb, edge_index):
    hcat = jnp.concatenate([user_emb, item_emb], axis=0)
    hpad = jnp.zeros((NP, D), jnp.float32).at[:N].set(hcat)
    hhalf = jnp.stack([hpad[:, :DH], hpad[:, DH:]])
    src2 = edge_index[0].reshape(NSUB, MIDC, INNER, K)
    dst2 = edge_index[1].reshape(NSUB, MIDC, INNER, K)
    out = _lightgcn_sc(src2, dst2, hhalf)
    full = jnp.concatenate([out[0, :N], out[1, :N]], axis=1)
    return full[:N_USERS], full[N_USERS:]


# speedup vs baseline: 7.3423x; 1.3595x over previous
import jax
import jax.numpy as jnp
from jax import lax
from jax.experimental import pallas as pl
from jax.experimental.pallas import tpu as pltpu
from jax.experimental.pallas import tpu_sc as plsc

N_USERS = 25000
N_ITEMS = 25000
N = N_USERS + N_ITEMS
NP = 50176
E = 800000
D = 64
DH = 32
LAYERS = 3

NSUB = 16
EPS = E // NSUB
K = 125
INNER = 8
MIDC = EPS // (INNER * K)
RPS = NP // NSUB
RC = 56
NCH = RPS // RC

_SC_PARAMS = pltpu.CompilerParams(
    needs_layout_passes=False, use_tc_tiling_on_sc=False)


def _body_deg(src2, dst2, degs, deg_sh, idx_v, zv, ones_v, sem):
    c = lax.axis_index("c")
    s = lax.axis_index("s")
    z16 = jnp.zeros((16,), jnp.float32)
    nbase = s * RPS

    def _z1(i, _):
        zv[pl.ds(i * 16, 16)] = z16
        return 0
    lax.fori_loop(0, RPS // 16, _z1, 0)

    def _o1(i, _):
        ones_v[pl.ds(i * 16, 16)] = jnp.ones((16,), jnp.float32)
        return 0
    lax.fori_loop(0, 8, _o1, 0)

    pltpu.sync_copy(zv, deg_sh.at[pl.ds(nbase, RPS)])
    plsc.subcore_barrier()

    ones_k = ones_v.at[pl.ds(0, K)]
    for cc, arr in ((0, src2), (1, dst2)):
        @pl.when(c == cc)
        def _():
            def _mid(m, _):
                pltpu.sync_copy(arr.at[s, m], idx_v)

                def _inner(j, _):
                    pltpu.sync_copy(ones_k, deg_sh.at[idx_v.at[j]], add=True)
                    return 0
                lax.fori_loop(0, INNER, _inner, 0)
                return 0
            lax.fori_loop(0, MIDC, _mid, 0)
    plsc.subcore_barrier()
    pltpu.sync_copy(deg_sh.at[pl.ds(nbase, RPS)], zv)
    pltpu.sync_copy(zv, degs.at[c, pl.ds(nbase, RPS)])


BR = 1568


def _body_norm(do_ref, di_ref, on_ref, in_ref, io_ref):
    on = lax.rsqrt(jnp.maximum(do_ref[...], 1.0))
    inn = lax.rsqrt(jnp.maximum(di_ref[...], 1.0))

    def expand(x):
        return jnp.concatenate(
            [jnp.broadcast_to(x[:, k:k + 1], (BR, DH)) for k in range(4)],
            axis=1)
    on_ref[...] = expand(on)
    in_ref[...] = expand(inn)
    io_ref[...] = expand(on * inn)


def _body_main(src2, dst2, hhalf, onorm, inorm, ionorm, out, tbl,
               agg_sh, idx_s, idx_d, g0, g1, agg_v, acc_v, tbl_v,
               n1_v, n2_v, zero2d, sem0, sem1):
    c = lax.axis_index("c")
    s = lax.axis_index("s")
    z16 = jnp.zeros((16,), jnp.float32)
    nbase = s * RPS

    def _z2(r, _):
        zero2d[r, pl.ds(0, 16)] = z16
        zero2d[r, pl.ds(16, 16)] = z16
        return 0
    lax.fori_loop(0, RC, _z2, 0)

    def _init_chunk(k, _):
        rows0 = nbase + k * RC
        pltpu.sync_copy(hhalf.at[c, pl.ds(rows0, RC)], agg_v)
        pltpu.sync_copy(onorm.at[pl.ds(rows0, RC)], n1_v)
        pltpu.sync_copy(agg_v, out.at[c, pl.ds(rows0, RC)])

        def _scale(r, _):
            tbl_v[r, pl.ds(0, 16)] = agg_v[r, pl.ds(0, 16)] * n1_v[r, pl.ds(0, 16)]
            tbl_v[r, pl.ds(16, 16)] = agg_v[r, pl.ds(16, 16)] * n1_v[r, pl.ds(16, 16)]
            return 0
        lax.fori_loop(0, RC, _scale, 0)
        pltpu.sync_copy(tbl_v, tbl.at[c, 0, pl.ds(rows0, RC)])
        return 0
    lax.fori_loop(0, NCH, _init_chunk, 0)

    gbufs = (g0, g1)
    sems = (sem0, sem1)
    for layer in range(1, LAYERS + 1):
        tp = (layer - 1) % 2
        tn = layer % 2
        last = layer == LAYERS

        def _zero_chunk(k, _):
            pltpu.sync_copy(zero2d, agg_sh.at[pl.ds(nbase + k * RC, RC)])
            return 0
        lax.fori_loop(0, NCH, _zero_chunk, 0)
        plsc.subcore_barrier()

        tcur = tbl.at[c, tp]

        def _edge_mid(m, _):
            pltpu.sync_copy(src2.at[s, m], idx_s)
            pltpu.sync_copy(dst2.at[s, m], idx_d)
            handles = [None, None]
            handles[0] = pltpu.async_copy(tcur.at[idx_s.at[0]], g0, sem0)
            for j in range(INNER):
                if j + 1 < INNER:
                    handles[(j + 1) % 2] = pltpu.async_copy(
                        tcur.at[idx_s.at[j + 1]], gbufs[(j + 1) % 2],
                        sems[(j + 1) % 2])
                handles[j % 2].wait()
                pltpu.sync_copy(gbufs[j % 2], agg_sh.at[idx_d.at[j]], add=True)
            return 0
        lax.fori_loop(0, MIDC, _edge_mid, 0)
        plsc.subcore_barrier()

        def _fin_chunk(k, _):
            rows0 = nbase + k * RC
            pltpu.sync_copy(agg_sh.at[pl.ds(rows0, RC)], agg_v)
            pltpu.sync_copy(inorm.at[pl.ds(rows0, RC)], n1_v)
            if not last:
                pltpu.sync_copy(ionorm.at[pl.ds(rows0, RC)], n2_v)
            pltpu.sync_copy(out.at[c, pl.ds(rows0, RC)], acc_v)

            def _fin(r, _):
                for half in (0, 16):
                    sl = pl.ds(half, 16)
                    h = agg_v[r, sl] * n1_v[r, sl]
                    a = acc_v[r, sl] + h
                    if last:
                        a = a * 0.25
                    acc_v[r, sl] = a
                    if not last:
                        tbl_v[r, sl] = agg_v[r, sl] * n2_v[r, sl]
                return 0
            lax.fori_loop(0, RC, _fin, 0)
            pltpu.sync_copy(acc_v, out.at[c, pl.ds(rows0, RC)])
            if not last:
                pltpu.sync_copy(tbl_v, tbl.at[c, tn, pl.ds(rows0, RC)])
            return 0
        lax.fori_loop(0, NCH, _fin_chunk, 0)


@jax.jit
def _lightgcn_sc(src2, dst2, hhalf):
    mesh = plsc.VectorSubcoreMesh(core_axis_name="c", subcore_axis_name="s")
    f32 = jnp.float32

    deg_run = pl.kernel(
        _body_deg,
        mesh=mesh,
        compiler_params=_SC_PARAMS,
        out_type=[jax.ShapeDtypeStruct((2, NP), f32)],
        scratch_types=[
            pltpu.VMEM_SHARED((NP,), f32),
            pltpu.VMEM((INNER, K), jnp.int32),
            pltpu.VMEM((RPS,), f32),
            pltpu.VMEM((128,), f32),
            pltpu.SemaphoreType.DMA,
        ],
    )
    [degs] = deg_run(src2, dst2)

    onorm, inorm, ionorm = pl.pallas_call(
        _body_norm,
        grid=(NP // 4 // BR,),
        in_specs=[pl.BlockSpec((BR, 4), lambda i: (i, 0))] * 2,
        out_specs=[pl.BlockSpec((BR, 128), lambda i: (i, 0))] * 3,
        out_shape=[jax.ShapeDtypeStruct((NP // 4, 128), f32)] * 3,
    )(degs[0].reshape(NP // 4, 4), degs[1].reshape(NP // 4, 4))
    onorm = onorm.reshape(NP, DH)
    inorm = inorm.reshape(NP, DH)
    ionorm = ionorm.reshape(NP, DH)

    main_run = pl.kernel(
        _body_main,
        mesh=mesh,
        compiler_params=_SC_PARAMS,
        out_type=[
            jax.ShapeDtypeStruct((2, NP, DH), f32),
            jax.ShapeDtypeStruct((2, 2, NP, DH), f32),
        ],
        scratch_types=[
            pltpu.VMEM_SHARED((NP, DH), f32),
            pltpu.VMEM((INNER, K), jnp.int32),
            pltpu.VMEM((INNER, K), jnp.int32),
            pltpu.VMEM((K, DH), f32),
            pltpu.VMEM((K, DH), f32),
            pltpu.VMEM((RC, DH), f32),
            pltpu.VMEM((RC, DH), f32),
            pltpu.VMEM((RC, DH), f32),
            pltpu.VMEM((RC, DH), f32),
            pltpu.VMEM((RC, DH), f32),
            pltpu.VMEM((RC, DH), f32),
            pltpu.SemaphoreType.DMA,
            pltpu.SemaphoreType.DMA,
        ],
    )
    out, _ = main_run(src2, dst2, hhalf, onorm, inorm, ionorm)
    return out


def kernel(user_emb, item_emb, edge_index):
    hcat = jnp.concatenate([user_emb, item_emb], axis=0)
    hpad = jnp.zeros((NP, D), jnp.float32).at[:N].set(hcat)
    hhalf = jnp.stack([hpad[:, :DH], hpad[:, DH:]])
    src2 = edge_index[0].reshape(NSUB, MIDC, INNER, K)
    dst2 = edge_index[1].reshape(NSUB, MIDC, INNER, K)
    out = _lightgcn_sc(src2, dst2, hhalf)
    full = jnp.concatenate([out[0, :N], out[1, :N]], axis=1)
    return full[:N_USERS], full[N_USERS:]


# speedup vs baseline: 7.7197x; 1.0514x over previous
import jax
import jax.numpy as jnp
from jax import lax
from jax.experimental import pallas as pl
from jax.experimental.pallas import tpu as pltpu
from jax.experimental.pallas import tpu_sc as plsc

N_USERS = 25000
N_ITEMS = 25000
N = N_USERS + N_ITEMS
NP = 50176
E = 800000
D = 64
DH = 32
LAYERS = 3

NSUB = 16
EPS = E // NSUB
K = 125
INNER = 8
MIDC = EPS // (INNER * K)
RPS = NP // NSUB
RC = 56
NCH = RPS // RC

_SC_PARAMS = pltpu.CompilerParams(
    needs_layout_passes=False, use_tc_tiling_on_sc=False)


def _body_deg(src2, dst2, degs, deg_sh, idx_v, zv, ones_v, sem):
    c = lax.axis_index("c")
    s = lax.axis_index("s")
    z16 = jnp.zeros((16,), jnp.float32)
    nbase = s * RPS

    def _z1(i, _):
        zv[pl.ds(i * 16, 16)] = z16
        return 0
    lax.fori_loop(0, RPS // 16, _z1, 0)

    def _o1(i, _):
        ones_v[pl.ds(i * 16, 16)] = jnp.ones((16,), jnp.float32)
        return 0
    lax.fori_loop(0, 8, _o1, 0)

    pltpu.sync_copy(zv, deg_sh.at[pl.ds(nbase, RPS)])
    plsc.subcore_barrier()

    ones_k = ones_v.at[pl.ds(0, K)]
    for cc, arr in ((0, src2), (1, dst2)):
        @pl.when(c == cc)
        def _():
            def _mid(m, _):
                pltpu.sync_copy(arr.at[s, m], idx_v)

                def _inner(j, _):
                    pltpu.sync_copy(ones_k, deg_sh.at[idx_v.at[j]], add=True)
                    return 0
                lax.fori_loop(0, INNER, _inner, 0)
                return 0
            lax.fori_loop(0, MIDC, _mid, 0)
    plsc.subcore_barrier()
    pltpu.sync_copy(deg_sh.at[pl.ds(nbase, RPS)], zv)
    pltpu.sync_copy(zv, degs.at[c, pl.ds(nbase, RPS)])


BR = 1568


def _body_norm(do_ref, di_ref, on_ref, in_ref, io_ref):
    on = lax.rsqrt(jnp.maximum(do_ref[...], 1.0))
    inn = lax.rsqrt(jnp.maximum(di_ref[...], 1.0))

    def expand(x):
        return jnp.concatenate(
            [jnp.broadcast_to(x[:, k:k + 1], (BR, DH)) for k in range(4)],
            axis=1)
    on_ref[...] = expand(on)
    in_ref[...] = expand(inn)
    io_ref[...] = expand(on * inn)


def _body_main(src2, dst2, hhalf, onorm, inorm, ionorm, out, tbl,
               agg_sh, idx_s, idx_d, g0, g1, g2, agg_v, acc_v, tbl_v,
               n1_v, n2_v, zero2d, gs0, gs1, gs2, ss0, ss1, ss2):
    c = lax.axis_index("c")
    s = lax.axis_index("s")
    z16 = jnp.zeros((16,), jnp.float32)
    nbase = s * RPS

    def _z2(r, _):
        zero2d[r, pl.ds(0, 16)] = z16
        zero2d[r, pl.ds(16, 16)] = z16
        return 0
    lax.fori_loop(0, RC, _z2, 0)

    def _init_chunk(k, _):
        rows0 = nbase + k * RC
        pltpu.sync_copy(hhalf.at[c, pl.ds(rows0, RC)], agg_v)
        pltpu.sync_copy(onorm.at[pl.ds(rows0, RC)], n1_v)
        pltpu.sync_copy(agg_v, out.at[c, pl.ds(rows0, RC)])

        def _scale(r, _):
            tbl_v[r, pl.ds(0, 16)] = agg_v[r, pl.ds(0, 16)] * n1_v[r, pl.ds(0, 16)]
            tbl_v[r, pl.ds(16, 16)] = agg_v[r, pl.ds(16, 16)] * n1_v[r, pl.ds(16, 16)]
            return 0
        lax.fori_loop(0, RC, _scale, 0)
        pltpu.sync_copy(tbl_v, tbl.at[c, 0, pl.ds(rows0, RC)])
        return 0
    lax.fori_loop(0, NCH, _init_chunk, 0)

    gbufs = (g0, g1, g2)
    gsems = (gs0, gs1, gs2)
    ssems = (ss0, ss1, ss2)
    NBUF = 3
    for layer in range(1, LAYERS + 1):
        tp = (layer - 1) % 2
        tn = layer % 2
        last = layer == LAYERS

        def _zero_chunk(k, _):
            pltpu.sync_copy(zero2d, agg_sh.at[pl.ds(nbase + k * RC, RC)])
            return 0
        lax.fori_loop(0, NCH, _zero_chunk, 0)
        plsc.subcore_barrier()

        tcur = tbl.at[c, tp]

        def _edge_mid(m, _):
            pltpu.sync_copy(src2.at[s, m], idx_s)
            pltpu.sync_copy(dst2.at[s, m], idx_d)
            gh = [None] * NBUF
            sh = [None] * NBUF
            gh[0] = pltpu.async_copy(tcur.at[idx_s.at[0]], gbufs[0], gsems[0])
            gh[1] = pltpu.async_copy(tcur.at[idx_s.at[1]], gbufs[1], gsems[1])
            for j in range(INNER):
                b = j % NBUF
                gh[b].wait()
                sh[b] = pltpu.async_copy(
                    gbufs[b], agg_sh.at[idx_d.at[j]], ssems[b], add=True)
                nj = j + 2
                if nj < INNER:
                    nb = nj % NBUF
                    if sh[nb] is not None:
                        sh[nb].wait()
                        sh[nb] = None
                    gh[nb] = pltpu.async_copy(
                        tcur.at[idx_s.at[nj]], gbufs[nb], gsems[nb])
            for b in range(NBUF):
                if sh[b] is not None:
                    sh[b].wait()
            return 0
        lax.fori_loop(0, MIDC, _edge_mid, 0)
        plsc.subcore_barrier()

        def _fin_chunk(k, _):
            rows0 = nbase + k * RC
            pltpu.sync_copy(agg_sh.at[pl.ds(rows0, RC)], agg_v)
            pltpu.sync_copy(inorm.at[pl.ds(rows0, RC)], n1_v)
            if not last:
                pltpu.sync_copy(ionorm.at[pl.ds(rows0, RC)], n2_v)
            pltpu.sync_copy(out.at[c, pl.ds(rows0, RC)], acc_v)

            def _fin(r, _):
                for half in (0, 16):
                    sl = pl.ds(half, 16)
                    h = agg_v[r, sl] * n1_v[r, sl]
                    a = acc_v[r, sl] + h
                    if last:
                        a = a * 0.25
                    acc_v[r, sl] = a
                    if not last:
                        tbl_v[r, sl] = agg_v[r, sl] * n2_v[r, sl]
                return 0
            lax.fori_loop(0, RC, _fin, 0)
            pltpu.sync_copy(acc_v, out.at[c, pl.ds(rows0, RC)])
            if not last:
                pltpu.sync_copy(tbl_v, tbl.at[c, tn, pl.ds(rows0, RC)])
            return 0
        lax.fori_loop(0, NCH, _fin_chunk, 0)


@jax.jit
def _lightgcn_sc(src2, dst2, hhalf):
    mesh = plsc.VectorSubcoreMesh(core_axis_name="c", subcore_axis_name="s")
    f32 = jnp.float32

    deg_run = pl.kernel(
        _body_deg,
        mesh=mesh,
        compiler_params=_SC_PARAMS,
        out_type=[jax.ShapeDtypeStruct((2, NP), f32)],
        scratch_types=[
            pltpu.VMEM_SHARED((NP,), f32),
            pltpu.VMEM((INNER, K), jnp.int32),
            pltpu.VMEM((RPS,), f32),
            pltpu.VMEM((128,), f32),
            pltpu.SemaphoreType.DMA,
        ],
    )
    [degs] = deg_run(src2, dst2)

    onorm, inorm, ionorm = pl.pallas_call(
        _body_norm,
        grid=(NP // 4 // BR,),
        in_specs=[pl.BlockSpec((BR, 4), lambda i: (i, 0))] * 2,
        out_specs=[pl.BlockSpec((BR, 128), lambda i: (i, 0))] * 3,
        out_shape=[jax.ShapeDtypeStruct((NP // 4, 128), f32)] * 3,
    )(degs[0].reshape(NP // 4, 4), degs[1].reshape(NP // 4, 4))
    onorm = onorm.reshape(NP, DH)
    inorm = inorm.reshape(NP, DH)
    ionorm = ionorm.reshape(NP, DH)

    main_run = pl.kernel(
        _body_main,
        mesh=mesh,
        compiler_params=_SC_PARAMS,
        out_type=[
            jax.ShapeDtypeStruct((2, NP, DH), f32),
            jax.ShapeDtypeStruct((2, 2, NP, DH), f32),
        ],
        scratch_types=[
            pltpu.VMEM_SHARED((NP, DH), f32),
            pltpu.VMEM((INNER, K), jnp.int32),
            pltpu.VMEM((INNER, K), jnp.int32),
            pltpu.VMEM((K, DH), f32),
            pltpu.VMEM((K, DH), f32),
            pltpu.VMEM((K, DH), f32),
            pltpu.VMEM((RC, DH), f32),
            pltpu.VMEM((RC, DH), f32),
            pltpu.VMEM((RC, DH), f32),
            pltpu.VMEM((RC, DH), f32),
            pltpu.VMEM((RC, DH), f32),
            pltpu.VMEM((RC, DH), f32),
            pltpu.SemaphoreType.DMA,
            pltpu.SemaphoreType.DMA,
            pltpu.SemaphoreType.DMA,
            pltpu.SemaphoreType.DMA,
            pltpu.SemaphoreType.DMA,
            pltpu.SemaphoreType.DMA,
        ],
    )
    out, _ = main_run(src2, dst2, hhalf, onorm, inorm, ionorm)
    return out


def kernel(user_emb, item_emb, edge_index):
    hcat = jnp.concatenate([user_emb, item_emb], axis=0)
    hpad = jnp.zeros((NP, D), jnp.float32).at[:N].set(hcat)
    hhalf = jnp.stack([hpad[:, :DH], hpad[:, DH:]])
    src2 = edge_index[0].reshape(NSUB, MIDC, INNER, K)
    dst2 = edge_index[1].reshape(NSUB, MIDC, INNER, K)
    out = _lightgcn_sc(src2, dst2, hhalf)
    full = jnp.concatenate([out[0, :N], out[1, :N]], axis=1)
    return full[:N_USERS], full[N_USERS:]


# speedup vs baseline: 9.5602x; 1.2384x over previous
import jax
import jax.numpy as jnp
from jax import lax
from jax.experimental import pallas as pl
from jax.experimental.pallas import tpu as pltpu
from jax.experimental.pallas import tpu_sc as plsc

N_USERS = 25000
N_ITEMS = 25000
N = N_USERS + N_ITEMS
NP = 50176
E = 800000
D = 64
DH = 32
LAYERS = 3

NSUB = 16
EPS = E // NSUB
K = 125
INNER = 8
MIDC = EPS // (INNER * K)
RPS = NP // NSUB
RC = 112
NCH = RPS // RC

_SC_PARAMS = pltpu.CompilerParams(
    needs_layout_passes=False, use_tc_tiling_on_sc=False)


def _body_deg(src2, dst2, degs, deg_sh, idx_v, zv, ones_v, sem):
    c = lax.axis_index("c")
    s = lax.axis_index("s")
    z16 = jnp.zeros((16,), jnp.float32)
    nbase = s * RPS

    def _z1(i, _):
        zv[pl.ds(i * 16, 16)] = z16
        return 0
    lax.fori_loop(0, RPS // 16, _z1, 0)

    def _o1(i, _):
        ones_v[pl.ds(i * 16, 16)] = jnp.ones((16,), jnp.float32)
        return 0
    lax.fori_loop(0, 8, _o1, 0)

    pltpu.sync_copy(zv, deg_sh.at[pl.ds(nbase, RPS)])
    plsc.subcore_barrier()

    ones_k = ones_v.at[pl.ds(0, K)]
    for cc, arr in ((0, src2), (1, dst2)):
        @pl.when(c == cc)
        def _():
            def _mid(m, _):
                pltpu.sync_copy(arr.at[s, m], idx_v)

                def _inner(j, _):
                    pltpu.sync_copy(ones_k, deg_sh.at[idx_v.at[j]], add=True)
                    return 0
                lax.fori_loop(0, INNER, _inner, 0)
                return 0
            lax.fori_loop(0, MIDC, _mid, 0)
    plsc.subcore_barrier()
    pltpu.sync_copy(deg_sh.at[pl.ds(nbase, RPS)], zv)
    pltpu.sync_copy(zv, degs.at[c, pl.ds(nbase, RPS)])


BR = 1568


def _body_norm(do_ref, di_ref, on_ref, in_ref, io_ref):
    on = lax.rsqrt(jnp.maximum(do_ref[...], 1.0))
    inn = lax.rsqrt(jnp.maximum(di_ref[...], 1.0))

    def expand(x):
        return jnp.concatenate(
            [jnp.broadcast_to(x[:, k:k + 1], (BR, DH)) for k in range(4)],
            axis=1)
    on_ref[...] = expand(on)
    in_ref[...] = expand(inn)
    io_ref[...] = expand(on * inn)


def _body_main(src2, dst2, hhalf, onorm, inorm, ionorm, out, tbl,
               agg_sh, idx_s, idx_d, g0, g1, g2, agg_v, acc_v,
               n1_v, n2_v, gs0, gs1, gs2, ss0, ss1, ss2, isem, osem):
    c = lax.axis_index("c")
    s = lax.axis_index("s")
    z16 = jnp.zeros((16,), jnp.float32)
    nbase = s * RPS

    def _init_chunk(k, _):
        rows0 = nbase + k * RC
        h1 = pltpu.async_copy(hhalf.at[c, pl.ds(rows0, RC)], acc_v, isem)
        h2 = pltpu.async_copy(onorm.at[pl.ds(rows0, RC)], n1_v, osem)
        h1.wait()
        h2.wait()
        o1 = pltpu.async_copy(acc_v, out.at[c, pl.ds(rows0, RC)], isem)

        def _scale(r, _):
            for half in (0, 16):
                sl = pl.ds(half, 16)
                agg_v[r, sl] = acc_v[r, sl] * n1_v[r, sl]
            return 0
        lax.fori_loop(0, RC, _scale, 0)
        o2 = pltpu.async_copy(agg_v, tbl.at[c, 0, pl.ds(rows0, RC)], osem)
        o1.wait()
        o2.wait()
        return 0
    lax.fori_loop(0, NCH, _init_chunk, 0)

    gbufs = (g0, g1, g2)
    gsems = (gs0, gs1, gs2)
    ssems = (ss0, ss1, ss2)
    NBUF = 3
    for layer in range(1, LAYERS + 1):
        tp = (layer - 1) % 2
        tn = layer % 2
        last = layer == LAYERS

        def _zb(r, _):
            n2_v[r, pl.ds(0, 16)] = z16
            n2_v[r, pl.ds(16, 16)] = z16
            return 0
        lax.fori_loop(0, RC, _zb, 0)

        def _zero_chunk(k, _):
            pltpu.sync_copy(n2_v, agg_sh.at[pl.ds(nbase + k * RC, RC)])
            return 0
        lax.fori_loop(0, NCH, _zero_chunk, 0)
        plsc.subcore_barrier()

        tcur = tbl.at[c, tp]

        def _edge_mid(m, _):
            pltpu.sync_copy(src2.at[s, m], idx_s)
            pltpu.sync_copy(dst2.at[s, m], idx_d)
            gh = [None] * NBUF
            sh = [None] * NBUF
            gh[0] = pltpu.async_copy(tcur.at[idx_s.at[0]], gbufs[0], gsems[0])
            gh[1] = pltpu.async_copy(tcur.at[idx_s.at[1]], gbufs[1], gsems[1])
            for j in range(INNER):
                b = j % NBUF
                gh[b].wait()
                sh[b] = pltpu.async_copy(
                    gbufs[b], agg_sh.at[idx_d.at[j]], ssems[b], add=True)
                nj = j + 2
                if nj < INNER:
                    nb = nj % NBUF
                    if sh[nb] is not None:
                        sh[nb].wait()
                        sh[nb] = None
                    gh[nb] = pltpu.async_copy(
                        tcur.at[idx_s.at[nj]], gbufs[nb], gsems[nb])
            for b in range(NBUF):
                if sh[b] is not None:
                    sh[b].wait()
            return 0
        lax.fori_loop(0, MIDC, _edge_mid, 0)
        plsc.subcore_barrier()

        def _fin_chunk(k, _):
            rows0 = nbase + k * RC
            h1 = pltpu.async_copy(agg_sh.at[pl.ds(rows0, RC)], agg_v, isem)
            h2 = pltpu.async_copy(inorm.at[pl.ds(rows0, RC)], n1_v, osem)
            h3 = pltpu.async_copy(out.at[c, pl.ds(rows0, RC)], acc_v, gs0)
            h4 = (pltpu.async_copy(ionorm.at[pl.ds(rows0, RC)], n2_v, gs1)
                  if not last else None)
            h1.wait()
            h2.wait()
            h3.wait()
            if h4 is not None:
                h4.wait()

            def _fin(r, _):
                for half in (0, 16):
                    sl = pl.ds(half, 16)
                    g = agg_v[r, sl]
                    a = acc_v[r, sl] + g * n1_v[r, sl]
                    if last:
                        a = a * 0.25
                    acc_v[r, sl] = a
                    if not last:
                        agg_v[r, sl] = g * n2_v[r, sl]
                return 0
            lax.fori_loop(0, RC, _fin, 0)
            o1 = pltpu.async_copy(acc_v, out.at[c, pl.ds(rows0, RC)], isem)
            o2 = (pltpu.async_copy(
                agg_v, tbl.at[c, tn, pl.ds(rows0, RC)], osem)
                if not last else None)
            o1.wait()
            if o2 is not None:
                o2.wait()
            return 0
        lax.fori_loop(0, NCH, _fin_chunk, 0)


@jax.jit
def _lightgcn_sc(src2, dst2, hhalf):
    mesh = plsc.VectorSubcoreMesh(core_axis_name="c", subcore_axis_name="s")
    f32 = jnp.float32

    deg_run = pl.kernel(
        _body_deg,
        mesh=mesh,
        compiler_params=_SC_PARAMS,
        out_type=[jax.ShapeDtypeStruct((2, NP), f32)],
        scratch_types=[
            pltpu.VMEM_SHARED((NP,), f32),
            pltpu.VMEM((INNER, K), jnp.int32),
            pltpu.VMEM((RPS,), f32),
            pltpu.VMEM((128,), f32),
            pltpu.SemaphoreType.DMA,
        ],
    )
    [degs] = deg_run(src2, dst2)

    onorm, inorm, ionorm = pl.pallas_call(
        _body_norm,
        grid=(NP // 4 // BR,),
        in_specs=[pl.BlockSpec((BR, 4), lambda i: (i, 0))] * 2,
        out_specs=[pl.BlockSpec((BR, 128), lambda i: (i, 0))] * 3,
        out_shape=[jax.ShapeDtypeStruct((NP // 4, 128), f32)] * 3,
    )(degs[0].reshape(NP // 4, 4), degs[1].reshape(NP // 4, 4))
    onorm = onorm.reshape(NP, DH)
    inorm = inorm.reshape(NP, DH)
    ionorm = ionorm.reshape(NP, DH)

    main_run = pl.kernel(
        _body_main,
        mesh=mesh,
        compiler_params=_SC_PARAMS,
        out_type=[
            jax.ShapeDtypeStruct((2, NP, DH), f32),
            jax.ShapeDtypeStruct((2, 2, NP, DH), f32),
        ],
        scratch_types=[
            pltpu.VMEM_SHARED((NP, DH), f32),
            pltpu.VMEM((INNER, K), jnp.int32),
            pltpu.VMEM((INNER, K), jnp.int32),
            pltpu.VMEM((K, DH), f32),
            pltpu.VMEM((K, DH), f32),
            pltpu.VMEM((K, DH), f32),
            pltpu.VMEM((RC, DH), f32),
            pltpu.VMEM((RC, DH), f32),
            pltpu.VMEM((RC, DH), f32),
            pltpu.VMEM((RC, DH), f32),
        ] + [pltpu.SemaphoreType.DMA] * 8,
    )
    out, _ = main_run(src2, dst2, hhalf, onorm, inorm, ionorm)
    return out


def kernel(user_emb, item_emb, edge_index):
    hcat = jnp.concatenate([user_emb, item_emb], axis=0)
    hpad = jnp.zeros((NP, D), jnp.float32).at[:N].set(hcat)
    hhalf = jnp.stack([hpad[:, :DH], hpad[:, DH:]])
    src2 = edge_index[0].reshape(NSUB, MIDC, INNER, K)
    dst2 = edge_index[1].reshape(NSUB, MIDC, INNER, K)
    out = _lightgcn_sc(src2, dst2, hhalf)
    full = jnp.concatenate([out[0, :N], out[1, :N]], axis=1)
    return full[:N_USERS], full[N_USERS:]


# speedup vs baseline: 10.9415x; 1.1445x over previous
import jax
import jax.numpy as jnp
from jax import lax
from jax.experimental import pallas as pl
from jax.experimental.pallas import tpu as pltpu
from jax.experimental.pallas import tpu_sc as plsc

N_USERS = 25000
N_ITEMS = 25000
N = N_USERS + N_ITEMS
NP = 50176
E = 800000
D = 64
DH = 32
LAYERS = 3

NSUB = 16
EPS = E // NSUB
K = 125
INNER = 8
MIDC = EPS // (INNER * K)
RPS = NP // NSUB
RC = 112
NCH = RPS // RC

_SC_PARAMS = pltpu.CompilerParams(
    needs_layout_passes=False, use_tc_tiling_on_sc=False)


def _body_deg(src2, dst2, degs, deg_sh, idx_v, zv, ones_v, sem):
    c = lax.axis_index("c")
    s = lax.axis_index("s")
    z16 = jnp.zeros((16,), jnp.float32)
    nbase = s * RPS

    def _z1(i, _):
        zv[pl.ds(i * 16, 16)] = z16
        return 0
    lax.fori_loop(0, RPS // 16, _z1, 0)

    def _o1(i, _):
        ones_v[pl.ds(i * 16, 16)] = jnp.ones((16,), jnp.float32)
        return 0
    lax.fori_loop(0, 8, _o1, 0)

    pltpu.sync_copy(zv, deg_sh.at[pl.ds(nbase, RPS)])
    plsc.subcore_barrier()

    ones_k = ones_v.at[pl.ds(0, K)]
    for cc, arr in ((0, src2), (1, dst2)):
        @pl.when(c == cc)
        def _():
            def _mid(m, _):
                pltpu.sync_copy(arr.at[s, m], idx_v)

                def _inner(j, _):
                    pltpu.sync_copy(ones_k, deg_sh.at[idx_v.at[j]], add=True)
                    return 0
                lax.fori_loop(0, INNER, _inner, 0)
                return 0
            lax.fori_loop(0, MIDC, _mid, 0)
    plsc.subcore_barrier()
    pltpu.sync_copy(deg_sh.at[pl.ds(nbase, RPS)], zv)
    pltpu.sync_copy(zv, degs.at[c, pl.ds(nbase, RPS)])


BR = 1568


def _body_norm(do_ref, di_ref, on_ref, in_ref, io_ref):
    on = lax.rsqrt(jnp.maximum(do_ref[...], 1.0))
    inn = lax.rsqrt(jnp.maximum(di_ref[...], 1.0))

    def expand(x):
        return jnp.concatenate(
            [jnp.broadcast_to(x[:, k:k + 1], (BR, DH)) for k in range(4)],
            axis=1)
    on_ref[...] = expand(on)
    in_ref[...] = expand(inn)
    io_ref[...] = expand(on * inn)


def _body_main(src2, dst2, hhalf, onorm, inorm, ionorm, out, tbl,
               agg_sh, idx_s, idx_d, idx_s2, idx_d2, g0, g1, g2, agg_v,
               acc_v, n1_v, n2_v, gs0, gs1, gs2, ss0, ss1, ss2, isem, osem):
    c = lax.axis_index("c")
    s = lax.axis_index("s")
    z16 = jnp.zeros((16,), jnp.float32)
    nbase = s * RPS

    def _init_chunk(k, _):
        rows0 = nbase + k * RC
        h1 = pltpu.async_copy(hhalf.at[c, pl.ds(rows0, RC)], acc_v, isem)
        h2 = pltpu.async_copy(onorm.at[pl.ds(rows0, RC)], n1_v, osem)
        h1.wait()
        h2.wait()
        o1 = pltpu.async_copy(acc_v, out.at[c, pl.ds(rows0, RC)], isem)

        def _scale(r, _):
            for half in (0, 16):
                sl = pl.ds(half, 16)
                agg_v[r, sl] = acc_v[r, sl] * n1_v[r, sl]
            return 0
        lax.fori_loop(0, RC, _scale, 0)
        o2 = pltpu.async_copy(agg_v, tbl.at[c, 0, pl.ds(rows0, RC)], osem)
        o1.wait()
        o2.wait()
        return 0
    lax.fori_loop(0, NCH, _init_chunk, 0)

    gbufs = (g0, g1, g2)
    gsems = (gs0, gs1, gs2)
    ssems = (ss0, ss1, ss2)
    NBUF = 3
    for layer in range(1, LAYERS + 1):
        tp = (layer - 1) % 2
        tn = layer % 2
        last = layer == LAYERS

        if layer == 1:
            def _zb(r, _):
                n2_v[r, pl.ds(0, 16)] = z16
                n2_v[r, pl.ds(16, 16)] = z16
                return 0
            lax.fori_loop(0, RC, _zb, 0)

            def _zero_chunk(k, _):
                pltpu.sync_copy(n2_v, agg_sh.at[pl.ds(nbase + k * RC, RC)])
                return 0
            lax.fori_loop(0, NCH, _zero_chunk, 0)
        plsc.subcore_barrier()

        tcur = tbl.at[c, tp]

        def _ring(is_, id_):
            gh = [None] * NBUF
            sh = [None] * NBUF
            gh[0] = pltpu.async_copy(tcur.at[is_.at[0]], gbufs[0], gsems[0])
            gh[1] = pltpu.async_copy(tcur.at[is_.at[1]], gbufs[1], gsems[1])
            for j in range(INNER):
                b = j % NBUF
                gh[b].wait()
                sh[b] = pltpu.async_copy(
                    gbufs[b], agg_sh.at[id_.at[j]], ssems[b], add=True)
                nj = j + 2
                if nj < INNER:
                    nb = nj % NBUF
                    if sh[nb] is not None:
                        sh[nb].wait()
                        sh[nb] = None
                    gh[nb] = pltpu.async_copy(
                        tcur.at[is_.at[nj]], gbufs[nb], gsems[nb])
            for b in range(NBUF):
                if sh[b] is not None:
                    sh[b].wait()

        pltpu.sync_copy(src2.at[s, 0], idx_s)
        pltpu.sync_copy(dst2.at[s, 0], idx_d)

        def _edge_pair(t, _):
            m0 = 2 * t
            hb1 = pltpu.async_copy(src2.at[s, m0 + 1], idx_s2, isem)
            hb2 = pltpu.async_copy(dst2.at[s, m0 + 1], idx_d2, osem)
            _ring(idx_s, idx_d)
            hb1.wait()
            hb2.wait()
            m2 = jnp.minimum(m0 + 2, MIDC - 1)
            ha1 = pltpu.async_copy(src2.at[s, m2], idx_s, isem)
            ha2 = pltpu.async_copy(dst2.at[s, m2], idx_d, osem)
            _ring(idx_s2, idx_d2)
            ha1.wait()
            ha2.wait()
            return 0
        lax.fori_loop(0, MIDC // 2, _edge_pair, 0)
        plsc.subcore_barrier()

        if not last:
            def _zg(r, _):
                g0[r, pl.ds(0, 16)] = z16
                g0[r, pl.ds(16, 16)] = z16
                return 0
            lax.fori_loop(0, RC, _zg, 0)

        def _fin_chunk(k, _):
            rows0 = nbase + k * RC
            h1 = pltpu.async_copy(agg_sh.at[pl.ds(rows0, RC)], agg_v, isem)
            h2 = pltpu.async_copy(inorm.at[pl.ds(rows0, RC)], n1_v, osem)
            h3 = pltpu.async_copy(out.at[c, pl.ds(rows0, RC)], acc_v, gs0)
            h4 = (pltpu.async_copy(ionorm.at[pl.ds(rows0, RC)], n2_v, gs1)
                  if not last else None)
            h1.wait()
            h5 = (pltpu.async_copy(
                g0.at[pl.ds(0, RC)], agg_sh.at[pl.ds(rows0, RC)], ss0)
                if not last else None)
            h2.wait()
            h3.wait()
            if h4 is not None:
                h4.wait()

            def _fin(r, _):
                for half in (0, 16):
                    sl = pl.ds(half, 16)
                    g = agg_v[r, sl]
                    a = acc_v[r, sl] + g * n1_v[r, sl]
                    if last:
                        a = a * 0.25
                    acc_v[r, sl] = a
                    if not last:
                        agg_v[r, sl] = g * n2_v[r, sl]
                return 0
            lax.fori_loop(0, RC, _fin, 0)
            o1 = pltpu.async_copy(acc_v, out.at[c, pl.ds(rows0, RC)], isem)
            o2 = (pltpu.async_copy(
                agg_v, tbl.at[c, tn, pl.ds(rows0, RC)], osem)
                if not last else None)
            o1.wait()
            if o2 is not None:
                o2.wait()
            if h5 is not None:
                h5.wait()
            return 0
        lax.fori_loop(0, NCH, _fin_chunk, 0)


@jax.jit
def _lightgcn_sc(src2, dst2, hhalf):
    mesh = plsc.VectorSubcoreMesh(core_axis_name="c", subcore_axis_name="s")
    f32 = jnp.float32

    deg_run = pl.kernel(
        _body_deg,
        mesh=mesh,
        compiler_params=_SC_PARAMS,
        out_type=[jax.ShapeDtypeStruct((2, NP), f32)],
        scratch_types=[
            pltpu.VMEM_SHARED((NP,), f32),
            pltpu.VMEM((INNER, K), jnp.int32),
            pltpu.VMEM((RPS,), f32),
            pltpu.VMEM((128,), f32),
            pltpu.SemaphoreType.DMA,
        ],
    )
    [degs] = deg_run(src2, dst2)

    onorm, inorm, ionorm = pl.pallas_call(
        _body_norm,
        grid=(NP // 4 // BR,),
        in_specs=[pl.BlockSpec((BR, 4), lambda i: (i, 0))] * 2,
        out_specs=[pl.BlockSpec((BR, 128), lambda i: (i, 0))] * 3,
        out_shape=[jax.ShapeDtypeStruct((NP // 4, 128), f32)] * 3,
    )(degs[0].reshape(NP // 4, 4), degs[1].reshape(NP // 4, 4))
    onorm = onorm.reshape(NP, DH)
    inorm = inorm.reshape(NP, DH)
    ionorm = ionorm.reshape(NP, DH)

    main_run = pl.kernel(
        _body_main,
        mesh=mesh,
        compiler_params=_SC_PARAMS,
        out_type=[
            jax.ShapeDtypeStruct((2, NP, DH), f32),
            jax.ShapeDtypeStruct((2, 2, NP, DH), f32),
        ],
        scratch_types=[
            pltpu.VMEM_SHARED((NP, DH), f32),
            pltpu.VMEM((INNER, K), jnp.int32),
            pltpu.VMEM((INNER, K), jnp.int32),
            pltpu.VMEM((INNER, K), jnp.int32),
            pltpu.VMEM((INNER, K), jnp.int32),
            pltpu.VMEM((K, DH), f32),
            pltpu.VMEM((K, DH), f32),
            pltpu.VMEM((K, DH), f32),
            pltpu.VMEM((RC, DH), f32),
            pltpu.VMEM((RC, DH), f32),
            pltpu.VMEM((RC, DH), f32),
            pltpu.VMEM((RC, DH), f32),
        ] + [pltpu.SemaphoreType.DMA] * 8,
    )
    out, _ = main_run(src2, dst2, hhalf, onorm, inorm, ionorm)
    return out


def kernel(user_emb, item_emb, edge_index):
    hcat = jnp.concatenate([user_emb, item_emb], axis=0)
    hpad = jnp.zeros((NP, D), jnp.float32).at[:N].set(hcat)
    hhalf = jnp.stack([hpad[:, :DH], hpad[:, DH:]])
    src2 = edge_index[0].reshape(NSUB, MIDC, INNER, K)
    dst2 = edge_index[1].reshape(NSUB, MIDC, INNER, K)
    out = _lightgcn_sc(src2, dst2, hhalf)
    full = jnp.concatenate([out[0, :N], out[1, :N]], axis=1)
    return full[:N_USERS], full[N_USERS:]
